# Initial kernel scaffold; baseline (speedup 1.0000x reference)
import jax
import jax.numpy as jnp
from jax.experimental import pallas as pl


def kernel(x, edge_index, W1, b1, W2, b2):
    raise NotImplementedError("write your pallas kernel here")


# speedup vs baseline: 24.6104x; 24.6104x over previous
import functools

import jax
import jax.numpy as jnp
from jax import lax
from jax.experimental import pallas as pl
from jax.experimental.pallas import tpu as pltpu
from jax.experimental.pallas import tpu_sc as plsc

NC = 2
NT = 16
CH = 128
KM = 8
HID = 64
NGROUP = 4
GW = 16


def _deg_body(n_acc, cpw, row2, ones2, degp, row_v, ones_v, zbuf, acc, sem):
    c = lax.axis_index("c")
    s = lax.axis_index("s")
    w = c * NT + s
    stripe = n_acc // NT

    def zb(i, _):
        zbuf[pl.ds(i * 16, 16)] = jnp.zeros((16,), jnp.float32)
        return 0

    lax.fori_loop(0, stripe // 16, zb, 0)
    pltpu.sync_copy(zbuf, acc.at[pl.ds(s * stripe, stripe)])
    plsc.subcore_barrier()

    cbase = w * cpw

    def macro(i, _):
        mb = cbase + i * KM
        pltpu.sync_copy(row2.at[pl.ds(mb, KM)], row_v)
        pltpu.sync_copy(ones2.at[pl.ds(mb, KM)], ones_v)
        for j in range(KM):
            pltpu.sync_copy(ones_v.at[j], acc.at[row_v.at[j]], add=True)
        return 0

    lax.fori_loop(0, cpw // KM, macro, 0)
    plsc.subcore_barrier()
    pltpu.sync_copy(acc.at[pl.ds(s * stripe, stripe)], zbuf)
    pltpu.sync_copy(zbuf, degp.at[pl.ds(c * n_acc + s * stripe, stripe)])


def _agg_body(np_, cpt, row2, col2, g0, g1, g2, g3, s0, s1, s2, s3,
              row_v, col_v, rows_v, stage_v, acc, sem_g, sem_s):
    c = lax.axis_index("c")
    s = lax.axis_index("s")
    rpt = np_ // NT
    sc = next(d for d in range(512, 0, -8) if rpt % d == 0)
    nchunk = rpt // sc
    g_refs = (g0, g1, g2, g3)
    s_refs = (s0, s1, s2, s3)
    stripe = s * rpt
    cbase = s * cpt

    for group in range(NGROUP):
        def run(gr=g_refs[group], sr=s_refs[group]):
            for k in range(nchunk):
                pltpu.sync_copy(gr.at[pl.ds(stripe + k * sc, sc)], stage_v)
                pltpu.sync_copy(stage_v, acc.at[pl.ds(stripe + k * sc, sc)])
            plsc.subcore_barrier()

            def macro(i, _):
                mb = cbase + i * KM
                pltpu.sync_copy(row2.at[pl.ds(mb, KM)], row_v)
                pltpu.sync_copy(col2.at[pl.ds(mb, KM)], col_v)
                descs = [pltpu.async_copy(gr.at[row_v.at[j]], rows_v.at[j], sem_g)
                         for j in range(KM)]
                for d in descs:
                    d.wait()
                descs = [pltpu.async_copy(rows_v.at[j], acc.at[col_v.at[j]],
                                          sem_s, add=True) for j in range(KM)]
                for d in descs:
                    d.wait()
                return 0

            lax.fori_loop(0, cpt // KM, macro, 0)
            plsc.subcore_barrier()
            for k in range(nchunk):
                pltpu.sync_copy(acc.at[pl.ds(stripe + k * sc, sc)], stage_v)
                pltpu.sync_copy(stage_v, sr.at[pl.ds(stripe + k * sc, sc)])

        pl.when(c == group // 2)(run)


def _lin1_body(x_ref, dp_ref, W1_ref, b1_ref, g0, g1, g2, g3, dis_ref):
    deg = dp_ref[0, :] + dp_ref[1, :] + 1.0
    dis = lax.rsqrt(deg)
    h = lax.dot_general(x_ref[...], W1_ref[...], (((1,), (1,)), ((), ())),
                        preferred_element_type=jnp.float32) + b1_ref[...]
    g = dis[:, None] * h
    g0[...] = g[:, 0:16]
    g1[...] = g[:, 16:32]
    g2[...] = g[:, 32:48]
    g3[...] = g[:, 48:64]
    dis_ref[...] = dis[:, None]


def _lin2_body(s0, s1, s2, s3, dis_ref, W2_ref, b2_ref, out_ref):
    dis = dis_ref[...]
    tot = None
    for r, sref in enumerate((s0, s1, s2, s3)):
        t = jnp.maximum(dis * sref[...], 0.0)
        p = lax.dot_general(t, W2_ref[...][:, r * GW:(r + 1) * GW],
                            (((1,), (1,)), ((), ())),
                            preferred_element_type=jnp.float32)
        tot = p if tot is None else tot + p
    out_ref[...] = tot + b2_ref[...]


def kernel(x, edge_index, W1, b1, W2, b2):
    n = x.shape[0]
    e = edge_index.shape[1]
    f32 = jnp.float32

    row = edge_index[0]
    col = edge_index[1]

    np_ = ((n + NT * 8 - 1) // (NT * 8)) * (NT * 8)
    quantum = NT * KM * CH
    e_pad = ((e + quantum - 1) // quantum) * quantum
    pad = e_pad - e
    pidx = jnp.arange(pad, dtype=jnp.int32)
    row_p = jnp.concatenate([row, (pidx * 977) % n])
    col_p = jnp.concatenate([col, np_ + (pidx % 8)])
    ones_p = jnp.concatenate([jnp.ones((e,), f32), jnp.zeros((pad,), f32)])
    row2 = row_p.reshape(-1, CH)
    col2 = col_p.reshape(-1, CH)
    ones2 = ones_p.reshape(-1, CH)
    x_p = jnp.pad(x, ((0, np_ - n), (0, 0)))

    n_acc_a = np_
    cpw = e_pad // (CH * NC * NT)
    stripe_a = n_acc_a // NT
    deg_k = pl.kernel(
        functools.partial(_deg_body, n_acc_a, cpw),
        out_type=jax.ShapeDtypeStruct((NC * n_acc_a,), f32),
        mesh=plsc.VectorSubcoreMesh(core_axis_name="c", subcore_axis_name="s"),
        compiler_params=pltpu.CompilerParams(use_tc_tiling_on_sc=False),
        scratch_types=[
            pltpu.VMEM((KM, CH), jnp.int32),
            pltpu.VMEM((KM, CH), f32),
            pltpu.VMEM((stripe_a,), f32),
            pltpu.VMEM_SHARED((n_acc_a,), f32),
            pltpu.SemaphoreType.DMA,
        ],
    )
    degp = deg_k(row2, ones2)
    dp = degp.reshape(NC, np_)

    bn = 4096
    nb = pl.cdiv(np_, bn)
    g_spec = pl.BlockSpec((bn, GW), lambda i: (i, 0))
    b_out = pl.pallas_call(
        _lin1_body,
        grid=(nb,),
        in_specs=[
            pl.BlockSpec((bn, x.shape[1]), lambda i: (i, 0)),
            pl.BlockSpec((NC, bn), lambda i: (0, i)),
            pl.BlockSpec(W1.shape, lambda i: (0, 0)),
            pl.BlockSpec((1, HID), lambda i: (0, 0)),
        ],
        out_specs=[g_spec, g_spec, g_spec, g_spec,
                   pl.BlockSpec((bn, 1), lambda i: (i, 0))],
        out_shape=[jax.ShapeDtypeStruct((np_, GW), f32) for _ in range(NGROUP)]
        + [jax.ShapeDtypeStruct((np_, 1), f32)],
    )(x_p, dp, W1, b1.reshape(1, HID))
    g0, g1, g2, g3, dis = b_out

    n_acc = np_ + 8
    cpt = e_pad // (CH * NT)
    agg_k = pl.kernel(
        functools.partial(_agg_body, np_, cpt),
        out_type=[jax.ShapeDtypeStruct((np_, GW), f32) for _ in range(NGROUP)],
        mesh=plsc.VectorSubcoreMesh(core_axis_name="c", subcore_axis_name="s"),
        compiler_params=pltpu.CompilerParams(use_tc_tiling_on_sc=False),
        scratch_types=[
            pltpu.VMEM((KM, CH), jnp.int32),
            pltpu.VMEM((KM, CH), jnp.int32),
            pltpu.VMEM((KM, CH, GW), f32),
            pltpu.VMEM((next(d for d in range(512, 0, -8) if (np_ // NT) % d == 0), GW), f32),
            pltpu.VMEM_SHARED((n_acc, GW), f32),
            pltpu.SemaphoreType.DMA,
            pltpu.SemaphoreType.DMA,
        ],
    )
    s0, s1, s2, s3 = agg_k(row2, col2, g0, g1, g2, g3)

    s_spec = pl.BlockSpec((bn, GW), lambda i: (i, 0))
    out = pl.pallas_call(
        _lin2_body,
        grid=(nb,),
        in_specs=[s_spec, s_spec, s_spec, s_spec,
                  pl.BlockSpec((bn, 1), lambda i: (i, 0)),
                  pl.BlockSpec(W2.shape, lambda i: (0, 0)),
                  pl.BlockSpec((1, W2.shape[0]), lambda i: (0, 0))],
        out_specs=pl.BlockSpec((bn, W2.shape[0]), lambda i: (i, 0)),
        out_shape=jax.ShapeDtypeStruct((n, W2.shape[0]), f32),
    )(s0, s1, s2, s3, dis, W2, b2.reshape(1, -1))
    return out


# speedup vs baseline: 32.0505x; 1.3023x over previous
import functools

import jax
import jax.numpy as jnp
from jax import lax
from jax.experimental import pallas as pl
from jax.experimental.pallas import tpu as pltpu
from jax.experimental.pallas import tpu_sc as plsc

NC = 2
NT = 16
CH = 128
KM = 8
HID = 64
NGROUP = 4
GW = 16


def _deg_body(n_acc, cpw, row2, ones2, degp, row_v, ones_v, zbuf, acc, sem):
    c = lax.axis_index("c")
    s = lax.axis_index("s")
    w = c * NT + s
    stripe = n_acc // NT

    def zb(i, _):
        zbuf[pl.ds(i * 16, 16)] = jnp.zeros((16,), jnp.float32)
        return 0

    lax.fori_loop(0, stripe // 16, zb, 0)
    pltpu.sync_copy(zbuf, acc.at[pl.ds(s * stripe, stripe)])
    plsc.subcore_barrier()

    cbase = w * cpw

    def macro(i, _):
        mb = cbase + i * KM
        pltpu.sync_copy(row2.at[pl.ds(mb, KM)], row_v)
        pltpu.sync_copy(ones2.at[pl.ds(mb, KM)], ones_v)
        for j in range(KM):
            pltpu.sync_copy(ones_v.at[j], acc.at[row_v.at[j]], add=True)
        return 0

    lax.fori_loop(0, cpw // KM, macro, 0)
    plsc.subcore_barrier()
    pltpu.sync_copy(acc.at[pl.ds(s * stripe, stripe)], zbuf)
    pltpu.sync_copy(zbuf, degp.at[pl.ds(c * n_acc + s * stripe, stripe)])


def _agg_body(np_, cpt, row2, col2, g0, g1, g2, g3, s0, s1, s2, s3,
              row_v, col_v, rows_v, stage_v, acc, sem_g, sem_s):
    c = lax.axis_index("c")
    s = lax.axis_index("s")
    rpt = np_ // NT
    sc = next(d for d in range(512, 0, -8) if rpt % d == 0)
    nchunk = rpt // sc
    g_refs = (g0, g1, g2, g3)
    s_refs = (s0, s1, s2, s3)
    stripe = s * rpt
    cbase = s * cpt

    for group in range(NGROUP):
        def run(gr=g_refs[group], sr=s_refs[group]):
            for k in range(nchunk):
                pltpu.sync_copy(gr.at[pl.ds(stripe + k * sc, sc)], stage_v)
                pltpu.sync_copy(stage_v, acc.at[pl.ds(stripe + k * sc, sc)])
            plsc.subcore_barrier()

            def macro(i, _):
                mb = cbase + i * KM
                pltpu.sync_copy(row2.at[pl.ds(mb, KM)], row_v)
                pltpu.sync_copy(col2.at[pl.ds(mb, KM)], col_v)
                descs = [pltpu.async_copy(gr.at[row_v.at[j]], rows_v.at[j], sem_g)
                         for j in range(KM)]
                for d in descs:
                    d.wait()
                descs = [pltpu.async_copy(rows_v.at[j], acc.at[col_v.at[j]],
                                          sem_s, add=True) for j in range(KM)]
                for d in descs:
                    d.wait()
                return 0

            lax.fori_loop(0, cpt // KM, macro, 0)
            plsc.subcore_barrier()
            for k in range(nchunk):
                pltpu.sync_copy(acc.at[pl.ds(stripe + k * sc, sc)], stage_v)
                pltpu.sync_copy(stage_v, sr.at[pl.ds(stripe + k * sc, sc)])

        pl.when(c == group // 2)(run)


def _lin1_body(xp_ref, d0_ref, d1_ref, W1P_ref, b1P_ref, P_ref,
               gw0, gw1, gw2, gw3, disl_ref):
    deg = d0_ref[...] + d1_ref[...] + 1.0
    disp = lax.rsqrt(deg)
    disl = jnp.dot(disp, P_ref[...], preferred_element_type=jnp.float32)
    hp = jnp.dot(xp_ref[...], W1P_ref[...],
                 preferred_element_type=jnp.float32) + b1P_ref[...]
    for r, ref in enumerate((gw0, gw1, gw2, gw3)):
        ref[...] = hp[:, r * 128:(r + 1) * 128] * disl
    disl_ref[...] = disl


def _lin2_body(sw0, sw1, sw2, sw3, disl_ref, Q_ref, b2P_ref, out_ref):
    disl = disl_ref[...]
    tot = None
    for r, sref in enumerate((sw0, sw1, sw2, sw3)):
        t = disl * jnp.maximum(sref[...], 0.0)
        p = jnp.dot(t, Q_ref[r * 128:(r + 1) * 128, :],
                    preferred_element_type=jnp.float32)
        tot = p if tot is None else tot + p
    out_ref[...] = tot + b2P_ref[...]


def kernel(x, edge_index, W1, b1, W2, b2):
    n = x.shape[0]
    e = edge_index.shape[1]
    f32 = jnp.float32

    row = edge_index[0]
    col = edge_index[1]

    np_ = ((n + NT * 8 - 1) // (NT * 8)) * (NT * 8)
    quantum = NT * KM * CH
    e_pad = ((e + quantum - 1) // quantum) * quantum
    pad = e_pad - e
    pidx = jnp.arange(pad, dtype=jnp.int32)
    row_p = jnp.concatenate([row, (pidx * 977) % n])
    col_p = jnp.concatenate([col, np_ + (pidx % 8)])
    ones_p = jnp.concatenate([jnp.ones((e,), f32), jnp.zeros((pad,), f32)])
    row2 = row_p.reshape(-1, CH)
    col2 = col_p.reshape(-1, CH)
    ones2 = ones_p.reshape(-1, CH)
    x_p = jnp.pad(x, ((0, np_ - n), (0, 0)))

    n_acc_a = np_
    cpw = e_pad // (CH * NC * NT)
    stripe_a = n_acc_a // NT
    deg_k = pl.kernel(
        functools.partial(_deg_body, n_acc_a, cpw),
        out_type=jax.ShapeDtypeStruct((NC * n_acc_a,), f32),
        mesh=plsc.VectorSubcoreMesh(core_axis_name="c", subcore_axis_name="s"),
        compiler_params=pltpu.CompilerParams(use_tc_tiling_on_sc=False),
        scratch_types=[
            pltpu.VMEM((KM, CH), jnp.int32),
            pltpu.VMEM((KM, CH), f32),
            pltpu.VMEM((stripe_a,), f32),
            pltpu.VMEM_SHARED((n_acc_a,), f32),
            pltpu.SemaphoreType.DMA,
        ],
    )
    degp = deg_k(row2, ones2)
    d0 = degp[:np_].reshape(np_ // 8, 8)
    d1 = degp[np_:].reshape(np_ // 8, 8)

    np8 = np_ // 8
    fin = x.shape[1]
    xp = x_p.reshape(np8, 8 * fin)
    eye8 = jnp.eye(8, dtype=f32)
    W1P = jnp.concatenate(
        [jnp.kron(eye8, W1[r * GW:(r + 1) * GW, :].T) for r in range(NGROUP)],
        axis=1)
    b1P = jnp.concatenate(
        [jnp.tile(b1[r * GW:(r + 1) * GW], 8) for r in range(NGROUP)]
    ).reshape(1, NGROUP * 128)
    P = jnp.kron(eye8, jnp.ones((1, GW), f32))
    bp = 512
    nb = pl.cdiv(np8, bp)
    gw_spec = pl.BlockSpec((bp, 128), lambda i: (i, 0))
    b_out = pl.pallas_call(
        _lin1_body,
        grid=(nb,),
        in_specs=[
            pl.BlockSpec((bp, 8 * fin), lambda i: (i, 0)),
            pl.BlockSpec((bp, 8), lambda i: (i, 0)),
            pl.BlockSpec((bp, 8), lambda i: (i, 0)),
            pl.BlockSpec((8 * fin, NGROUP * 128), lambda i: (0, 0)),
            pl.BlockSpec((1, NGROUP * 128), lambda i: (0, 0)),
            pl.BlockSpec((8, 128), lambda i: (0, 0)),
        ],
        out_specs=[gw_spec] * (NGROUP + 1),
        out_shape=[jax.ShapeDtypeStruct((np8, 128), f32)
                   for _ in range(NGROUP + 1)],
    )(xp, d0, d1, W1P, b1P, P)
    gw0, gw1, gw2, gw3, disl = b_out
    g0, g1, g2, g3 = (gw.reshape(np_, GW) for gw in (gw0, gw1, gw2, gw3))

    n_acc = np_ + 8
    cpt = e_pad // (CH * NT)
    agg_k = pl.kernel(
        functools.partial(_agg_body, np_, cpt),
        out_type=[jax.ShapeDtypeStruct((np_, GW), f32) for _ in range(NGROUP)],
        mesh=plsc.VectorSubcoreMesh(core_axis_name="c", subcore_axis_name="s"),
        compiler_params=pltpu.CompilerParams(use_tc_tiling_on_sc=False),
        scratch_types=[
            pltpu.VMEM((KM, CH), jnp.int32),
            pltpu.VMEM((KM, CH), jnp.int32),
            pltpu.VMEM((KM, CH, GW), f32),
            pltpu.VMEM((next(d for d in range(512, 0, -8) if (np_ // NT) % d == 0), GW), f32),
            pltpu.VMEM_SHARED((n_acc, GW), f32),
            pltpu.SemaphoreType.DMA,
            pltpu.SemaphoreType.DMA,
        ],
    )
    s0, s1, s2, s3 = agg_k(row2, col2, g0, g1, g2, g3)
    sw = [sr.reshape(np8, 128) for sr in (s0, s1, s2, s3)]

    oc = W2.shape[0]
    Q = jnp.concatenate(
        [jnp.kron(eye8, W2[:, r * GW:(r + 1) * GW].T) for r in range(NGROUP)],
        axis=0)
    b2P = jnp.tile(b2, 8).reshape(1, 8 * oc)
    out_pk = pl.pallas_call(
        _lin2_body,
        grid=(nb,),
        in_specs=[gw_spec] * 4 + [
            gw_spec,
            pl.BlockSpec((NGROUP * 128, 8 * oc), lambda i: (0, 0)),
            pl.BlockSpec((1, 8 * oc), lambda i: (0, 0))],
        out_specs=pl.BlockSpec((bp, 8 * oc), lambda i: (i, 0)),
        out_shape=jax.ShapeDtypeStruct((np8, 8 * oc), f32),
    )(*sw, disl, Q, b2P)
    return out_pk.reshape(np_, oc)[:n]


# speedup vs baseline: 32.0612x; 1.0003x over previous
import functools

import jax
import jax.numpy as jnp
from jax import lax
from jax.experimental import pallas as pl
from jax.experimental.pallas import tpu as pltpu
from jax.experimental.pallas import tpu_sc as plsc

NC = 2
NT = 16
CH = 128
KM = 8
HID = 64
NGROUP = 4
GW = 16


def _deg_body(n_acc, cpw, row2, ones2, degp, row_v, ones_v, zbuf, acc, sem):
    c = lax.axis_index("c")
    s = lax.axis_index("s")
    w = c * NT + s
    stripe = n_acc // NT

    def zb(i, _):
        zbuf[pl.ds(i * 16, 16)] = jnp.zeros((16,), jnp.float32)
        return 0

    lax.fori_loop(0, stripe // 16, zb, 0)
    pltpu.sync_copy(zbuf, acc.at[pl.ds(s * stripe, stripe)])
    plsc.subcore_barrier()

    cbase = w * cpw

    def macro(i, _):
        mb = cbase + i * KM
        pltpu.sync_copy(row2.at[pl.ds(mb, KM)], row_v)
        pltpu.sync_copy(ones2.at[pl.ds(mb, KM)], ones_v)
        for j in range(KM):
            pltpu.sync_copy(ones_v.at[j], acc.at[row_v.at[j]], add=True)
        return 0

    lax.fori_loop(0, cpw // KM, macro, 0)
    plsc.subcore_barrier()
    pltpu.sync_copy(acc.at[pl.ds(s * stripe, stripe)], zbuf)
    pltpu.sync_copy(zbuf, degp.at[pl.ds(c * n_acc + s * stripe, stripe)])


def _agg_body(np_, cpt, row1, col1, g0, g1, g2, g3, s0, s1, s2, s3,
              row_v, col_v, rows_v, stage_v, acc, sem_g, sem_s):
    c = lax.axis_index("c")
    s = lax.axis_index("s")
    rpt = np_ // NT
    sc = next(d for d in range(512, 0, -8) if rpt % d == 0)
    nchunk = rpt // sc
    g_refs = (g0, g1, g2, g3)
    s_refs = (s0, s1, s2, s3)
    stripe = s * rpt
    cbase = s * cpt

    for group in range(NGROUP):
        def run(gr=g_refs[group], sr=s_refs[group]):
            for k in range(nchunk):
                pltpu.sync_copy(gr.at[pl.ds(stripe + k * sc, sc)], stage_v)
                pltpu.sync_copy(stage_v, acc.at[pl.ds(stripe + k * sc, sc)])
            plsc.subcore_barrier()

            def macro(i, _):
                eb = (cbase + i * KM) * CH
                pltpu.sync_copy(row1.at[pl.ds(eb, KM * CH)], row_v)
                pltpu.sync_copy(col1.at[pl.ds(eb, KM * CH)], col_v)
                pltpu.async_copy(gr.at[row_v], rows_v, sem_g).wait()
                pltpu.async_copy(rows_v, acc.at[col_v], sem_s, add=True).wait()
                return 0

            lax.fori_loop(0, cpt // KM, macro, 0)
            plsc.subcore_barrier()
            for k in range(nchunk):
                pltpu.sync_copy(acc.at[pl.ds(stripe + k * sc, sc)], stage_v)
                pltpu.sync_copy(stage_v, sr.at[pl.ds(stripe + k * sc, sc)])

        pl.when(c == group // 2)(run)


def _lin1_body(xp_ref, d0_ref, d1_ref, W1P_ref, b1P_ref, P_ref,
               gw0, gw1, gw2, gw3, disl_ref):
    deg = d0_ref[...] + d1_ref[...] + 1.0
    disp = lax.rsqrt(deg)
    disl = jnp.dot(disp, P_ref[...], preferred_element_type=jnp.float32)
    hp = jnp.dot(xp_ref[...], W1P_ref[...],
                 preferred_element_type=jnp.float32) + b1P_ref[...]
    for r, ref in enumerate((gw0, gw1, gw2, gw3)):
        ref[...] = hp[:, r * 128:(r + 1) * 128] * disl
    disl_ref[...] = disl


def _lin2_body(sw0, sw1, sw2, sw3, disl_ref, Q_ref, b2P_ref, out_ref):
    disl = disl_ref[...]
    tot = None
    for r, sref in enumerate((sw0, sw1, sw2, sw3)):
        t = disl * jnp.maximum(sref[...], 0.0)
        p = jnp.dot(t, Q_ref[r * 128:(r + 1) * 128, :],
                    preferred_element_type=jnp.float32)
        tot = p if tot is None else tot + p
    out_ref[...] = tot + b2P_ref[...]


def kernel(x, edge_index, W1, b1, W2, b2):
    n = x.shape[0]
    e = edge_index.shape[1]
    f32 = jnp.float32

    row = edge_index[0]
    col = edge_index[1]

    np_ = ((n + NT * 8 - 1) // (NT * 8)) * (NT * 8)
    quantum = NT * KM * CH
    e_pad = ((e + quantum - 1) // quantum) * quantum
    pad = e_pad - e
    pidx = jnp.arange(pad, dtype=jnp.int32)
    row_p = jnp.concatenate([row, (pidx * 977) % n])
    col_p = jnp.concatenate([col, np_ + (pidx % 8)])
    ones_p = jnp.concatenate([jnp.ones((e,), f32), jnp.zeros((pad,), f32)])
    row2 = row_p.reshape(-1, CH)
    col2 = col_p.reshape(-1, CH)
    ones2 = ones_p.reshape(-1, CH)
    x_p = jnp.pad(x, ((0, np_ - n), (0, 0)))

    n_acc_a = np_
    cpw = e_pad // (CH * NC * NT)
    stripe_a = n_acc_a // NT
    deg_k = pl.kernel(
        functools.partial(_deg_body, n_acc_a, cpw),
        out_type=jax.ShapeDtypeStruct((NC * n_acc_a,), f32),
        mesh=plsc.VectorSubcoreMesh(core_axis_name="c", subcore_axis_name="s"),
        compiler_params=pltpu.CompilerParams(use_tc_tiling_on_sc=False),
        scratch_types=[
            pltpu.VMEM((KM, CH), jnp.int32),
            pltpu.VMEM((KM, CH), f32),
            pltpu.VMEM((stripe_a,), f32),
            pltpu.VMEM_SHARED((n_acc_a,), f32),
            pltpu.SemaphoreType.DMA,
        ],
    )
    degp = deg_k(row2, ones2)
    d0 = degp[:np_].reshape(np_ // 8, 8)
    d1 = degp[np_:].reshape(np_ // 8, 8)

    np8 = np_ // 8
    fin = x.shape[1]
    xp = x_p.reshape(np8, 8 * fin)
    eye8 = jnp.eye(8, dtype=f32)
    W1P = jnp.concatenate(
        [jnp.kron(eye8, W1[r * GW:(r + 1) * GW, :].T) for r in range(NGROUP)],
        axis=1)
    b1P = jnp.concatenate(
        [jnp.tile(b1[r * GW:(r + 1) * GW], 8) for r in range(NGROUP)]
    ).reshape(1, NGROUP * 128)
    P = jnp.kron(eye8, jnp.ones((1, GW), f32))
    bp = 512
    nb = pl.cdiv(np8, bp)
    gw_spec = pl.BlockSpec((bp, 128), lambda i: (i, 0))
    b_out = pl.pallas_call(
        _lin1_body,
        grid=(nb,),
        in_specs=[
            pl.BlockSpec((bp, 8 * fin), lambda i: (i, 0)),
            pl.BlockSpec((bp, 8), lambda i: (i, 0)),
            pl.BlockSpec((bp, 8), lambda i: (i, 0)),
            pl.BlockSpec((8 * fin, NGROUP * 128), lambda i: (0, 0)),
            pl.BlockSpec((1, NGROUP * 128), lambda i: (0, 0)),
            pl.BlockSpec((8, 128), lambda i: (0, 0)),
        ],
        out_specs=[gw_spec] * (NGROUP + 1),
        out_shape=[jax.ShapeDtypeStruct((np8, 128), f32)
                   for _ in range(NGROUP + 1)],
    )(xp, d0, d1, W1P, b1P, P)
    gw0, gw1, gw2, gw3, disl = b_out
    g0, g1, g2, g3 = (gw.reshape(np_, GW) for gw in (gw0, gw1, gw2, gw3))

    n_acc = np_ + 8
    cpt = e_pad // (CH * NT)
    agg_k = pl.kernel(
        functools.partial(_agg_body, np_, cpt),
        out_type=[jax.ShapeDtypeStruct((np_, GW), f32) for _ in range(NGROUP)],
        mesh=plsc.VectorSubcoreMesh(core_axis_name="c", subcore_axis_name="s"),
        compiler_params=pltpu.CompilerParams(use_tc_tiling_on_sc=False),
        scratch_types=[
            pltpu.VMEM((KM * CH,), jnp.int32),
            pltpu.VMEM((KM * CH,), jnp.int32),
            pltpu.VMEM((KM * CH, GW), f32),
            pltpu.VMEM((next(d for d in range(512, 0, -8) if (np_ // NT) % d == 0), GW), f32),
            pltpu.VMEM_SHARED((n_acc, GW), f32),
            pltpu.SemaphoreType.DMA,
            pltpu.SemaphoreType.DMA,
        ],
    )
    s0, s1, s2, s3 = agg_k(row_p, col_p, g0, g1, g2, g3)
    sw = [sr.reshape(np8, 128) for sr in (s0, s1, s2, s3)]

    oc = W2.shape[0]
    Q = jnp.concatenate(
        [jnp.kron(eye8, W2[:, r * GW:(r + 1) * GW].T) for r in range(NGROUP)],
        axis=0)
    b2P = jnp.tile(b2, 8).reshape(1, 8 * oc)
    out_pk = pl.pallas_call(
        _lin2_body,
        grid=(nb,),
        in_specs=[gw_spec] * 4 + [
            gw_spec,
            pl.BlockSpec((NGROUP * 128, 8 * oc), lambda i: (0, 0)),
            pl.BlockSpec((1, 8 * oc), lambda i: (0, 0))],
        out_specs=pl.BlockSpec((bp, 8 * oc), lambda i: (i, 0)),
        out_shape=jax.ShapeDtypeStruct((np8, 8 * oc), f32),
    )(*sw, disl, Q, b2P)
    return out_pk.reshape(np_, oc)[:n]


# speedup vs baseline: 41.2075x; 1.2853x over previous
import functools

import jax
import jax.numpy as jnp
from jax import lax
from jax.experimental import pallas as pl
from jax.experimental.pallas import tpu as pltpu
from jax.experimental.pallas import tpu_sc as plsc

NC = 2
NT = 16
CH = 128
KM = 8
HID = 64
NGROUP = 4
GW = 16


def _deg_body(n_acc, cpw, row2, ones2, degp, row_v, ones_v, zbuf, acc, sem):
    c = lax.axis_index("c")
    s = lax.axis_index("s")
    w = c * NT + s
    stripe = n_acc // NT

    def zb(i, _):
        zbuf[pl.ds(i * 16, 16)] = jnp.zeros((16,), jnp.float32)
        return 0

    lax.fori_loop(0, stripe // 16, zb, 0)
    pltpu.sync_copy(zbuf, acc.at[pl.ds(s * stripe, stripe)])
    plsc.subcore_barrier()

    cbase = w * cpw

    def macro(i, _):
        mb = cbase + i * KM
        pltpu.sync_copy(row2.at[pl.ds(mb, KM)], row_v)
        pltpu.sync_copy(ones2.at[pl.ds(mb, KM)], ones_v)
        for j in range(KM):
            pltpu.sync_copy(ones_v.at[j], acc.at[row_v.at[j]], add=True)
        return 0

    lax.fori_loop(0, cpw // KM, macro, 0)
    plsc.subcore_barrier()
    pltpu.sync_copy(acc.at[pl.ds(s * stripe, stripe)], zbuf)
    pltpu.sync_copy(zbuf, degp.at[pl.ds(c * n_acc + s * stripe, stripe)])


EC = 512
MPB = 8


def _agg_body(np_, nblk, row1, col3, g0, g1, g2, g3, s0, s1, s2, s3,
              row_i, col_i, rows_a, rows_b, stage_v, acc,
              sem_ga, sem_gb, sem_sa, sem_sb):
    c = lax.axis_index("c")
    s = lax.axis_index("s")
    rpt = np_ // NT
    scn = next(d for d in range(224, 0, -8) if rpt % d == 0)
    nchunk = rpt // scn
    g_refs = (g0, g1, g2, g3)
    s_refs = (s0, s1, s2, s3)
    stripe = s * rpt
    bufs = ((rows_a, sem_ga, sem_sa), (rows_b, sem_gb, sem_sb))

    for group in range(NGROUP):
        def run(gr=g_refs[group], sr=s_refs[group]):
            for k in range(nchunk):
                pltpu.sync_copy(gr.at[pl.ds(stripe + k * scn, scn)], stage_v)
                pltpu.sync_copy(stage_v, acc.at[pl.ds(stripe + k * scn, scn)])
            plsc.subcore_barrier()

            def fire_g(buf, m):
                pltpu.async_copy(gr.at[row_i.at[pl.ds(m * EC, EC)]], buf[0], buf[1])

            def wait_g(buf, m):
                pltpu.make_async_copy(gr.at[row_i.at[pl.ds(m * EC, EC)]], buf[0], buf[1]).wait()

            def fire_s(buf, m):
                pltpu.async_copy(buf[0], acc.at[col_i.at[m]], buf[2], add=True)

            def drain_s(buf, m):
                pltpu.make_async_copy(buf[0], acc.at[col_i.at[m]], buf[2]).wait()

            def blk(b, _):
                blkid = s * nblk + b
                pltpu.sync_copy(row1.at[pl.ds(blkid * (MPB * EC), MPB * EC)], row_i)
                pltpu.sync_copy(col3.at[blkid], col_i)
                for m in range(MPB):
                    buf = bufs[m % 2]
                    if m >= 2:
                        drain_s(buf, m - 2)
                    fire_g(buf, m)
                    if m >= 1:
                        pb = bufs[(m - 1) % 2]
                        wait_g(pb, m - 1)
                        fire_s(pb, m - 1)
                lb = bufs[(MPB - 1) % 2]
                wait_g(lb, MPB - 1)
                fire_s(lb, MPB - 1)
                drain_s(bufs[(MPB - 2) % 2], MPB - 2)
                drain_s(lb, MPB - 1)
                return 0

            lax.fori_loop(0, nblk, blk, 0)
            plsc.subcore_barrier()
            for k in range(nchunk):
                pltpu.sync_copy(acc.at[pl.ds(stripe + k * scn, scn)], stage_v)
                pltpu.sync_copy(stage_v, sr.at[pl.ds(stripe + k * scn, scn)])

        pl.when(c == group // 2)(run)


def _lin1_body(xp_ref, d0_ref, d1_ref, W1P_ref, b1P_ref, P_ref,
               gw0, gw1, gw2, gw3, disl_ref):
    deg = d0_ref[...] + d1_ref[...] + 1.0
    disp = lax.rsqrt(deg)
    disl = jnp.dot(disp, P_ref[...], preferred_element_type=jnp.float32)
    hp = jnp.dot(xp_ref[...], W1P_ref[...],
                 preferred_element_type=jnp.float32) + b1P_ref[...]
    for r, ref in enumerate((gw0, gw1, gw2, gw3)):
        ref[...] = hp[:, r * 128:(r + 1) * 128] * disl
    disl_ref[...] = disl


def _lin2_body(sw0, sw1, sw2, sw3, disl_ref, Q_ref, b2P_ref, out_ref):
    disl = disl_ref[...]
    tot = None
    for r, sref in enumerate((sw0, sw1, sw2, sw3)):
        t = disl * jnp.maximum(sref[...], 0.0)
        p = jnp.dot(t, Q_ref[r * 128:(r + 1) * 128, :],
                    preferred_element_type=jnp.float32)
        tot = p if tot is None else tot + p
    out_ref[...] = tot + b2P_ref[...]


def kernel(x, edge_index, W1, b1, W2, b2):
    n = x.shape[0]
    e = edge_index.shape[1]
    f32 = jnp.float32

    row = edge_index[0]
    col = edge_index[1]

    np_ = ((n + NT * 8 - 1) // (NT * 8)) * (NT * 8)
    quantum = NT * MPB * EC
    e_pad = ((e + quantum - 1) // quantum) * quantum
    pad = e_pad - e
    pidx = jnp.arange(pad, dtype=jnp.int32)
    row_p = jnp.concatenate([row, (pidx * 977) % n])
    col_p = jnp.concatenate([col, n + (pidx % (np_ - n))])
    ones_p = jnp.concatenate([jnp.ones((e,), f32), jnp.zeros((pad,), f32)])
    row2 = row_p.reshape(-1, CH)
    col2 = col_p.reshape(-1, CH)
    ones2 = ones_p.reshape(-1, CH)
    x_p = jnp.pad(x, ((0, np_ - n), (0, 0)))

    n_acc_a = np_
    cpw = e_pad // (CH * NC * NT)
    stripe_a = n_acc_a // NT
    deg_k = pl.kernel(
        functools.partial(_deg_body, n_acc_a, cpw),
        out_type=jax.ShapeDtypeStruct((NC * n_acc_a,), f32),
        mesh=plsc.VectorSubcoreMesh(core_axis_name="c", subcore_axis_name="s"),
        compiler_params=pltpu.CompilerParams(use_tc_tiling_on_sc=False),
        scratch_types=[
            pltpu.VMEM((KM, CH), jnp.int32),
            pltpu.VMEM((KM, CH), f32),
            pltpu.VMEM((stripe_a,), f32),
            pltpu.VMEM_SHARED((n_acc_a,), f32),
            pltpu.SemaphoreType.DMA,
        ],
    )
    degp = deg_k(row2, ones2)
    d0 = degp[:np_].reshape(np_ // 8, 8)
    d1 = degp[np_:].reshape(np_ // 8, 8)

    np8 = np_ // 8
    fin = x.shape[1]
    xp = x_p.reshape(np8, 8 * fin)
    eye8 = jnp.eye(8, dtype=f32)
    W1P = jnp.concatenate(
        [jnp.kron(eye8, W1[r * GW:(r + 1) * GW, :].T) for r in range(NGROUP)],
        axis=1)
    b1P = jnp.concatenate(
        [jnp.tile(b1[r * GW:(r + 1) * GW], 8) for r in range(NGROUP)]
    ).reshape(1, NGROUP * 128)
    P = jnp.kron(eye8, jnp.ones((1, GW), f32))
    bp = 512
    nb = pl.cdiv(np8, bp)
    gw_spec = pl.BlockSpec((bp, 128), lambda i: (i, 0))
    b_out = pl.pallas_call(
        _lin1_body,
        grid=(nb,),
        in_specs=[
            pl.BlockSpec((bp, 8 * fin), lambda i: (i, 0)),
            pl.BlockSpec((bp, 8), lambda i: (i, 0)),
            pl.BlockSpec((bp, 8), lambda i: (i, 0)),
            pl.BlockSpec((8 * fin, NGROUP * 128), lambda i: (0, 0)),
            pl.BlockSpec((1, NGROUP * 128), lambda i: (0, 0)),
            pl.BlockSpec((8, 128), lambda i: (0, 0)),
        ],
        out_specs=[gw_spec] * (NGROUP + 1),
        out_shape=[jax.ShapeDtypeStruct((np8, 128), f32)
                   for _ in range(NGROUP + 1)],
    )(xp, d0, d1, W1P, b1P, P)
    gw0, gw1, gw2, gw3, disl = b_out
    g0, g1, g2, g3 = (gw.reshape(np_, GW) for gw in (gw0, gw1, gw2, gw3))

    col3 = col_p.reshape(-1, MPB, EC)
    nblk = e_pad // (NT * MPB * EC)
    agg_k = pl.kernel(
        functools.partial(_agg_body, np_, nblk),
        out_type=[jax.ShapeDtypeStruct((np_, GW), f32) for _ in range(NGROUP)],
        mesh=plsc.VectorSubcoreMesh(core_axis_name="c", subcore_axis_name="s"),
        compiler_params=pltpu.CompilerParams(use_tc_tiling_on_sc=False),
        scratch_types=[
            pltpu.VMEM((MPB * EC,), jnp.int32),
            pltpu.VMEM((MPB, EC), jnp.int32),
            pltpu.VMEM((EC, GW), f32),
            pltpu.VMEM((EC, GW), f32),
            pltpu.VMEM((next(d for d in range(224, 0, -8) if (np_ // NT) % d == 0), GW), f32),
            pltpu.VMEM_SHARED((np_, GW), f32),
            pltpu.SemaphoreType.DMA,
            pltpu.SemaphoreType.DMA,
            pltpu.SemaphoreType.DMA,
            pltpu.SemaphoreType.DMA,
        ],
    )
    s0, s1, s2, s3 = agg_k(row_p, col3, g0, g1, g2, g3)
    sw = [sr.reshape(np8, 128) for sr in (s0, s1, s2, s3)]

    oc = W2.shape[0]
    Q = jnp.concatenate(
        [jnp.kron(eye8, W2[:, r * GW:(r + 1) * GW].T) for r in range(NGROUP)],
        axis=0)
    b2P = jnp.tile(b2, 8).reshape(1, 8 * oc)
    out_pk = pl.pallas_call(
        _lin2_body,
        grid=(nb,),
        in_specs=[gw_spec] * 4 + [
            gw_spec,
            pl.BlockSpec((NGROUP * 128, 8 * oc), lambda i: (0, 0)),
            pl.BlockSpec((1, 8 * oc), lambda i: (0, 0))],
        out_specs=pl.BlockSpec((bp, 8 * oc), lambda i: (i, 0)),
        out_shape=jax.ShapeDtypeStruct((np8, 8 * oc), f32),
    )(*sw, disl, Q, b2P)
    return out_pk.reshape(np_, oc)[:n]


# speedup vs baseline: 46.0169x; 1.1167x over previous
import functools

import jax
import jax.numpy as jnp
from jax import lax
from jax.experimental import pallas as pl
from jax.experimental.pallas import tpu as pltpu
from jax.experimental.pallas import tpu_sc as plsc

NC = 2
NT = 16
HID = 64
NGROUP = 4
GW = 16


ECA = 512
MPBA = 4


def _deg_body(n_acc, nblka, row3a, degp, ri_a, ri_b, ones_v, zbuf, acc,
              sem_a, sem_b):
    c = lax.axis_index("c")
    s = lax.axis_index("s")
    w = c * NT + s
    stripe = n_acc // NT

    def zb(i, _):
        zbuf[pl.ds(i * 16, 16)] = jnp.zeros((16,), jnp.float32)
        return 0

    lax.fori_loop(0, stripe // 16, zb, 0)

    def ob(i, _):
        ones_v[pl.ds(i * 16, 16)] = jnp.ones((16,), jnp.float32)
        return 0

    lax.fori_loop(0, ECA // 16, ob, 0)
    pltpu.sync_copy(zbuf, acc.at[pl.ds(s * stripe, stripe)])
    plsc.subcore_barrier()

    bufs = ((ri_a, sem_a), (ri_b, sem_b))

    def blk(b, _):
        def do(buf):
            ri, sem = buf

            @pl.when(b >= 2)
            def _():
                for m in range(MPBA):
                    pltpu.make_async_copy(ones_v, acc.at[ri.at[m]], sem).wait()

            pltpu.sync_copy(row3a.at[w * nblka + b], ri)
            for m in range(MPBA):
                pltpu.async_copy(ones_v, acc.at[ri.at[m]], sem, add=True)

        pl.when(b % 2 == 0)(lambda: do(bufs[0]))
        pl.when(b % 2 == 1)(lambda: do(bufs[1]))
        return 0

    lax.fori_loop(0, nblka, blk, 0)
    for ri, sem in bufs:
        for m in range(MPBA):
            pltpu.make_async_copy(ones_v, acc.at[ri.at[m]], sem).wait()
    plsc.subcore_barrier()
    pltpu.sync_copy(acc.at[pl.ds(s * stripe, stripe)], zbuf)
    pltpu.sync_copy(zbuf, degp.at[pl.ds(c * n_acc + s * stripe, stripe)])


EC = 512
MPB = 8


def _agg_body(np_, nblk, row3, col3, g0, g1, g2, g3, s0, s1, s2, s3,
              row_i, col_i, rows_a, rows_b, acc,
              sem_ga, sem_gb, sem_sa, sem_sb):
    c = lax.axis_index("c")
    s = lax.axis_index("s")
    rpt = np_ // NT
    g_refs = (g0, g1, g2, g3)
    s_refs = (s0, s1, s2, s3)
    stripe = s * rpt
    bufs = ((rows_a, sem_ga, sem_sa), (rows_b, sem_gb, sem_sb))

    for group in range(NGROUP):
        def run(gr=g_refs[group], sr=s_refs[group]):
            pltpu.sync_copy(gr.at[pl.ds(stripe, rpt)], acc.at[pl.ds(stripe, rpt)])
            plsc.subcore_barrier()

            def fire_g(buf, m):
                pltpu.async_copy(gr.at[row_i.at[m // MPBA, m % MPBA]], buf[0], buf[1])

            def wait_g(buf, m):
                pltpu.make_async_copy(gr.at[row_i.at[m // MPBA, m % MPBA]], buf[0], buf[1]).wait()

            def fire_s(buf, m):
                pltpu.async_copy(buf[0], acc.at[col_i.at[m // MPBA, m % MPBA]], buf[2], add=True)

            def drain_s(buf, m):
                pltpu.make_async_copy(buf[0], acc.at[col_i.at[m // MPBA, m % MPBA]], buf[2]).wait()

            def blk(b, _):
                blkid = s * nblk + b
                pltpu.sync_copy(row3.at[pl.ds(2 * blkid, 2)], row_i)
                pltpu.sync_copy(col3.at[pl.ds(2 * blkid, 2)], col_i)
                for m in range(MPB):
                    buf = bufs[m % 2]
                    if m >= 2:
                        drain_s(buf, m - 2)
                    fire_g(buf, m)
                    if m >= 1:
                        pb = bufs[(m - 1) % 2]
                        wait_g(pb, m - 1)
                        fire_s(pb, m - 1)
                lb = bufs[(MPB - 1) % 2]
                wait_g(lb, MPB - 1)
                fire_s(lb, MPB - 1)
                drain_s(bufs[(MPB - 2) % 2], MPB - 2)
                drain_s(lb, MPB - 1)
                return 0

            lax.fori_loop(0, nblk, blk, 0)
            plsc.subcore_barrier()
            pltpu.sync_copy(acc.at[pl.ds(stripe, rpt)], sr.at[pl.ds(stripe, rpt)])

        pl.when(c == group // 2)(run)


def _lin1_body(xp_ref, d0_ref, d1_ref, W1P_ref, b1P_ref, P_ref,
               gw0, gw1, gw2, gw3, disl_ref):
    deg = d0_ref[...] + d1_ref[...] + 1.0
    disp = lax.rsqrt(deg)
    disl = jnp.dot(disp, P_ref[...], preferred_element_type=jnp.float32)
    hp = jnp.dot(xp_ref[...], W1P_ref[...],
                 preferred_element_type=jnp.float32) + b1P_ref[...]
    for r, ref in enumerate((gw0, gw1, gw2, gw3)):
        ref[...] = hp[:, r * 128:(r + 1) * 128] * disl
    disl_ref[...] = disl


def _lin2_body(sw0, sw1, sw2, sw3, disl_ref, Q_ref, b2P_ref, out_ref):
    disl = disl_ref[...]
    tot = None
    for r, sref in enumerate((sw0, sw1, sw2, sw3)):
        t = disl * jnp.maximum(sref[...], 0.0)
        p = jnp.dot(t, Q_ref[r * 128:(r + 1) * 128, :],
                    preferred_element_type=jnp.float32)
        tot = p if tot is None else tot + p
    out_ref[...] = tot + b2P_ref[...]


def kernel(x, edge_index, W1, b1, W2, b2):
    n = x.shape[0]
    e = edge_index.shape[1]
    f32 = jnp.float32

    row = edge_index[0]
    col = edge_index[1]

    np_ = ((n + NT * 8 - 1) // (NT * 8)) * (NT * 8)
    quantum = NT * MPB * EC
    e_pad = ((e + quantum - 1) // quantum) * quantum
    pad = e_pad - e
    pidx = jnp.arange(pad, dtype=jnp.int32)
    jnk = n + (pidx % (np_ - n))
    row_p = jnp.concatenate([row, jnk])
    col_p = jnp.concatenate([col, jnk])
    x_p = jnp.pad(x, ((0, np_ - n), (0, 0)))

    n_acc_a = np_
    row3a = row_p.reshape(-1, MPBA, ECA)
    nblka = e_pad // (NC * NT * MPBA * ECA)
    stripe_a = n_acc_a // NT
    deg_k = pl.kernel(
        functools.partial(_deg_body, n_acc_a, nblka),
        out_type=jax.ShapeDtypeStruct((NC * n_acc_a,), f32),
        mesh=plsc.VectorSubcoreMesh(core_axis_name="c", subcore_axis_name="s"),
        compiler_params=pltpu.CompilerParams(use_tc_tiling_on_sc=False),
        scratch_types=[
            pltpu.VMEM((MPBA, ECA), jnp.int32),
            pltpu.VMEM((MPBA, ECA), jnp.int32),
            pltpu.VMEM((ECA,), f32),
            pltpu.VMEM((stripe_a,), f32),
            pltpu.VMEM_SHARED((n_acc_a,), f32),
            pltpu.SemaphoreType.DMA,
            pltpu.SemaphoreType.DMA,
        ],
    )
    degp = deg_k(row3a)
    d0 = degp[:np_].reshape(np_ // 8, 8)
    d1 = degp[np_:].reshape(np_ // 8, 8)

    np8 = np_ // 8
    fin = x.shape[1]
    xp = x_p.reshape(np8, 8 * fin)
    eye8 = jnp.eye(8, dtype=f32)
    W1P = jnp.concatenate(
        [jnp.kron(eye8, W1[r * GW:(r + 1) * GW, :].T) for r in range(NGROUP)],
        axis=1)
    b1P = jnp.concatenate(
        [jnp.tile(b1[r * GW:(r + 1) * GW], 8) for r in range(NGROUP)]
    ).reshape(1, NGROUP * 128)
    P = jnp.kron(eye8, jnp.ones((1, GW), f32))
    bp = 512
    nb = pl.cdiv(np8, bp)
    gw_spec = pl.BlockSpec((bp, 128), lambda i: (i, 0))
    b_out = pl.pallas_call(
        _lin1_body,
        grid=(nb,),
        in_specs=[
            pl.BlockSpec((bp, 8 * fin), lambda i: (i, 0)),
            pl.BlockSpec((bp, 8), lambda i: (i, 0)),
            pl.BlockSpec((bp, 8), lambda i: (i, 0)),
            pl.BlockSpec((8 * fin, NGROUP * 128), lambda i: (0, 0)),
            pl.BlockSpec((1, NGROUP * 128), lambda i: (0, 0)),
            pl.BlockSpec((8, 128), lambda i: (0, 0)),
        ],
        out_specs=[gw_spec] * (NGROUP + 1),
        out_shape=[jax.ShapeDtypeStruct((np8, 128), f32)
                   for _ in range(NGROUP + 1)],
    )(xp, d0, d1, W1P, b1P, P)
    gw0, gw1, gw2, gw3, disl = b_out
    g0, g1, g2, g3 = (gw.reshape(np_, GW) for gw in (gw0, gw1, gw2, gw3))

    col3 = col_p.reshape(-1, MPBA, ECA)
    nblk = e_pad // (NT * MPB * EC)
    agg_k = pl.kernel(
        functools.partial(_agg_body, np_, nblk),
        out_type=[jax.ShapeDtypeStruct((np_, GW), f32) for _ in range(NGROUP)],
        mesh=plsc.VectorSubcoreMesh(core_axis_name="c", subcore_axis_name="s"),
        compiler_params=pltpu.CompilerParams(use_tc_tiling_on_sc=False),
        scratch_types=[
            pltpu.VMEM((2, MPBA, ECA), jnp.int32),
            pltpu.VMEM((2, MPBA, ECA), jnp.int32),
            pltpu.VMEM((EC, GW), f32),
            pltpu.VMEM((EC, GW), f32),
            pltpu.VMEM_SHARED((np_, GW), f32),
            pltpu.SemaphoreType.DMA,
            pltpu.SemaphoreType.DMA,
            pltpu.SemaphoreType.DMA,
            pltpu.SemaphoreType.DMA,
        ],
    )
    s0, s1, s2, s3 = agg_k(row3a, col3, g0, g1, g2, g3)
    sw = [sr.reshape(np8, 128) for sr in (s0, s1, s2, s3)]

    oc = W2.shape[0]
    Q = jnp.concatenate(
        [jnp.kron(eye8, W2[:, r * GW:(r + 1) * GW].T) for r in range(NGROUP)],
        axis=0)
    b2P = jnp.tile(b2, 8).reshape(1, 8 * oc)
    out_pk = pl.pallas_call(
        _lin2_body,
        grid=(nb,),
        in_specs=[gw_spec] * 4 + [
            gw_spec,
            pl.BlockSpec((NGROUP * 128, 8 * oc), lambda i: (0, 0)),
            pl.BlockSpec((1, 8 * oc), lambda i: (0, 0))],
        out_specs=pl.BlockSpec((bp, 8 * oc), lambda i: (i, 0)),
        out_shape=jax.ShapeDtypeStruct((np8, 8 * oc), f32),
    )(*sw, disl, Q, b2P)
    return out_pk[:n // 8].reshape(n, oc)
